# baseline (speedup 1.0000x reference)
import jax
import jax.numpy as jnp
from jax.experimental import pallas as pl


def _add_body(x_ref, e_ref, o_ref):
    o_ref[...] = x_ref[...] + e_ref[...]


def kernel(inputs, embeddings):
    B, S, D = inputs.shape
    pos = embeddings[:S]
    SBLK = 512
    grid = (S // SBLK, B)
    return pl.pallas_call(
        _add_body,
        grid=grid,
        in_specs=[
            pl.BlockSpec((1, SBLK, D), lambda i, j: (j, i, 0)),
            pl.BlockSpec((SBLK, D), lambda i, j: (i, 0)),
        ],
        out_specs=pl.BlockSpec((1, SBLK, D), lambda i, j: (j, i, 0)),
        out_shape=jax.ShapeDtypeStruct((B, S, D), inputs.dtype),
    )(inputs, pos)


# speedup vs baseline: 1.1518x; 1.1518x over previous
import jax
import jax.numpy as jnp
from jax.experimental import pallas as pl


def _add_body(x_ref, e_ref, o_ref):
    o_ref[...] = x_ref[...] + e_ref[...]


def kernel(inputs, embeddings):
    B, S, D = inputs.shape
    pos = embeddings[:S]
    SBLK = 512
    grid = (S // SBLK,)
    return pl.pallas_call(
        _add_body,
        grid=grid,
        in_specs=[
            pl.BlockSpec((B, SBLK, D), lambda i: (0, i, 0)),
            pl.BlockSpec((SBLK, D), lambda i: (i, 0)),
        ],
        out_specs=pl.BlockSpec((B, SBLK, D), lambda i: (0, i, 0)),
        out_shape=jax.ShapeDtypeStruct((B, S, D), inputs.dtype),
    )(inputs, pos)
